# baseline (speedup 1.0000x reference)
import jax
import jax.numpy as jnp
from jax import lax
from jax.experimental import pallas as pl
from jax.experimental.pallas import tpu as pltpu
from jax.experimental.pallas import tpu_sc as plsc

_N_EXPERTS = 8
_TOP_K = 2
_LB_WEIGHT = 0.01
_NEG_BIG = -1e30


def _lane_gather(v, idx):
    return lax.gather(
        v, idx[:, None],
        lax.GatherDimensionNumbers(
            offset_dims=(), collapsed_slice_dims=(0,), start_index_map=(0,)),
        (1,), mode=lax.GatherScatterMode.PROMISE_IN_BOUNDS)


def _logits_body(x_ref, gw_ref, lt_ref):
    lt_ref[...] = lax.dot_general(
        gw_ref[...], x_ref[...], (((1,), (1,)), ((), ())),
        preferred_element_type=jnp.float32)


def _make_sc_router(n_tok):
    info = plsc.get_sparse_core_info()
    nc, ns, nl = info.num_cores, info.num_subcores, info.num_lanes
    nw = nc * ns
    tpw = n_tok // nw
    ngrp = tpw // nl
    E = _N_EXPERTS

    mesh = plsc.VectorSubcoreMesh(core_axis_name="c", subcore_axis_name="s")

    def body(lt_hbm, w_hbm, i_hbm, fp_hbm, l_v, w_v, i_v, st_v):
        wid = lax.axis_index("s") * nc + lax.axis_index("c")
        base = wid * tpw
        pltpu.sync_copy(lt_hbm.at[:, pl.ds(base, tpw)], l_v)
        lane = lax.iota(jnp.int32, nl)

        def grp(g, accs):
            ls = [l_v[e, pl.ds(g * nl, nl)] for e in range(E)]
            m1 = ls[0]
            i1 = jnp.zeros((nl,), jnp.int32)
            for e in range(1, E):
                c = ls[e] > m1
                m1 = jnp.where(c, ls[e], m1)
                i1 = jnp.where(c, e, i1)
            m2 = jnp.full((nl,), _NEG_BIG, jnp.float32)
            i2 = jnp.zeros((nl,), jnp.int32)
            for e in range(E):
                c = (i1 != e) & (ls[e] > m2)
                m2 = jnp.where(c, ls[e], m2)
                i2 = jnp.where(c, e, i2)
            r = jnp.exp(m2 - m1)
            w1 = 1.0 / (1.0 + r)
            w2 = r / (1.0 + r)
            half = lane >> 1
            even = (lane & 1) == 0
            wa = jnp.where(even, _lane_gather(w1, half), _lane_gather(w2, half))
            wb = jnp.where(even, _lane_gather(w1, half + (nl // 2)),
                           _lane_gather(w2, half + (nl // 2)))
            ia = jnp.where(even, _lane_gather(i1, half), _lane_gather(i2, half))
            ib = jnp.where(even, _lane_gather(i1, half + (nl // 2)),
                           _lane_gather(i2, half + (nl // 2)))
            w_v[pl.ds(g * 2 * nl, nl)] = wa
            w_v[pl.ds(g * 2 * nl + nl, nl)] = wb
            i_v[pl.ds(g * 2 * nl, nl)] = ia
            i_v[pl.ds(g * 2 * nl + nl, nl)] = ib
            ts = [jnp.exp(ls[e] - m1) for e in range(E)]
            denom = ts[0]
            for e in range(1, E):
                denom = denom + ts[e]
            inv = 1.0 / denom
            out = []
            for e in range(E):
                out.append(accs[e] + jnp.where((i1 == e) | (i2 == e), 1.0, 0.0))
            for e in range(E):
                out.append(accs[E + e] + ts[e] * inv)
            return tuple(out)

        zero = jnp.zeros((nl,), jnp.float32)
        accs = lax.fori_loop(0, ngrp, grp, tuple([zero] * (2 * E)))
        pltpu.sync_copy(w_v, w_hbm.at[pl.ds(2 * base, 2 * tpw)])
        pltpu.sync_copy(i_v, i_hbm.at[pl.ds(2 * base, 2 * tpw)])
        for rix in range(2 * E):
            st_v[rix, :] = accs[rix]
        pltpu.sync_copy(st_v, fp_hbm.at[pl.ds(wid * 2 * E, 2 * E)])

    out_type = [
        jax.ShapeDtypeStruct((_TOP_K * n_tok,), jnp.float32),
        jax.ShapeDtypeStruct((_TOP_K * n_tok,), jnp.int32),
        jax.ShapeDtypeStruct((nw * 2 * E, nl), jnp.float32),
    ]
    scratch_types = [
        pltpu.VMEM((E, tpw), jnp.float32),
        pltpu.VMEM((_TOP_K * tpw,), jnp.float32),
        pltpu.VMEM((_TOP_K * tpw,), jnp.int32),
        pltpu.VMEM((2 * E, nl), jnp.float32),
    ]
    return pl.kernel(body, mesh=mesh, out_type=out_type,
                     scratch_types=scratch_types), nw


def _make_aux_body(n_tok, nrow, nl):
    E = _N_EXPERTS

    def aux_body(fp_ref, aux_ref):
        a = fp_ref[...]
        rmod = lax.broadcasted_iota(jnp.int32, (nrow, nl), 0) % (2 * E)
        s = jnp.float32(0.0)
        for e in range(E):
            fs = jnp.sum(jnp.where(rmod == e, a, 0.0))
            ps = jnp.sum(jnp.where(rmod == E + e, a, 0.0))
            s = s + fs * ps
        n = jnp.float32(n_tok)
        aux_ref[...] = (E * _LB_WEIGHT * s / (n * n)).reshape(1, 1)

    return aux_body


def kernel(x, gate_w):
    b, s, d = x.shape
    n_tok = b * s
    xf = x.reshape(n_tok, d)
    blk = 1024
    grid = n_tok // blk

    lt = pl.pallas_call(
        _logits_body,
        grid=(grid,),
        in_specs=[
            pl.BlockSpec((blk, d), lambda i: (i, 0)),
            pl.BlockSpec((_N_EXPERTS, d), lambda i: (0, 0)),
        ],
        out_specs=pl.BlockSpec((_N_EXPERTS, blk), lambda i: (0, i)),
        out_shape=jax.ShapeDtypeStruct((_N_EXPERTS, n_tok), jnp.float32),
    )(xf, gate_w)

    sc_router, nw = _make_sc_router(n_tok)
    w_flat, i_flat, fp = sc_router(lt)

    nrow = nw * 2 * _N_EXPERTS
    nl = fp.shape[1]
    aux = pl.pallas_call(
        _make_aux_body(n_tok, nrow, nl),
        out_shape=jax.ShapeDtypeStruct((1, 1), jnp.float32),
    )(fp)

    top_k_weights = w_flat.reshape(b, s, _TOP_K)
    top_k_indices = i_flat.reshape(b, s, _TOP_K)
    return (top_k_weights, top_k_indices, aux[0, 0])


# speedup vs baseline: 1.3188x; 1.3188x over previous
import jax
import jax.numpy as jnp
from jax import lax
from jax.experimental import pallas as pl

_N_EXPERTS = 8
_TOP_K = 2
_LB_WEIGHT = 0.01


def _router_body(x_ref, gw_ref, w_ref, i_ref, facc_ref, pacc_ref, aux_ref):
    i = pl.program_id(0)
    nsteps = pl.num_programs(0)
    xb = x_ref[...]
    gw = gw_ref[...]
    l = lax.dot_general(
        xb, gw, (((1,), (1,)), ((), ())),
        preferred_element_type=jnp.float32)
    blk = l.shape[0]
    E = _N_EXPERTS
    e_iota = lax.broadcasted_iota(jnp.int32, (blk, E), 1)

    m1 = jnp.max(l, axis=1, keepdims=True)
    i1 = jnp.min(jnp.where(l == m1, e_iota, E), axis=1, keepdims=True)
    lm = jnp.where(e_iota == i1, -jnp.inf, l)
    m2 = jnp.max(lm, axis=1, keepdims=True)
    i2 = jnp.min(jnp.where(lm == m2, e_iota, E), axis=1, keepdims=True)

    r = jnp.exp(m2 - m1)
    w1 = 1.0 / (1.0 + r)
    w2 = r / (1.0 + r)
    w_ref[...] = jnp.concatenate([w1, w2], axis=1)
    i_ref[...] = jnp.concatenate([i1, i2], axis=1)

    t = jnp.exp(l - m1)
    denom = jnp.sum(t, axis=1, keepdims=True)
    probs = t / denom
    pc = jnp.sum(probs, axis=0, keepdims=True)
    mask = (e_iota == i1) | (e_iota == i2)
    fc = jnp.sum(jnp.where(mask, 1.0, 0.0), axis=0, keepdims=True)

    @pl.when(i == 0)
    def _init():
        facc_ref[...] = jnp.zeros_like(facc_ref)
        pacc_ref[...] = jnp.zeros_like(pacc_ref)

    facc_ref[...] += jnp.broadcast_to(fc, facc_ref.shape)
    pacc_ref[...] += jnp.broadcast_to(pc, pacc_ref.shape)

    @pl.when(i == nsteps - 1)
    def _fin():
        f = facc_ref[0:1, :]
        p = pacc_ref[0:1, :]
        s = jnp.sum(f * p)
        n_tok = jnp.float32(nsteps * blk)
        aux_ref[...] = (E * _LB_WEIGHT * s / (n_tok * n_tok)).reshape(1, 1)


def kernel(x, gate_w):
    b, s, d = x.shape
    n_tok = b * s
    xf = x.reshape(n_tok, d)
    blk = 1024
    grid = n_tok // blk
    E = _N_EXPERTS

    out_shapes = (
        jax.ShapeDtypeStruct((n_tok, _TOP_K), jnp.float32),
        jax.ShapeDtypeStruct((n_tok, _TOP_K), jnp.int32),
        jax.ShapeDtypeStruct((8, E), jnp.float32),
        jax.ShapeDtypeStruct((8, E), jnp.float32),
        jax.ShapeDtypeStruct((1, 1), jnp.float32),
    )
    w, it, _, _, aux = pl.pallas_call(
        _router_body,
        grid=(grid,),
        in_specs=[
            pl.BlockSpec((blk, d), lambda i: (i, 0)),
            pl.BlockSpec((E, d), lambda i: (0, 0)),
        ],
        out_specs=[
            pl.BlockSpec((blk, _TOP_K), lambda i: (i, 0)),
            pl.BlockSpec((blk, _TOP_K), lambda i: (i, 0)),
            pl.BlockSpec((8, E), lambda i: (0, 0)),
            pl.BlockSpec((8, E), lambda i: (0, 0)),
            pl.BlockSpec((1, 1), lambda i: (0, 0)),
        ],
        out_shape=out_shapes,
    )(xf, gate_w)

    top_k_weights = w.reshape(b, s, _TOP_K)
    top_k_indices = it.reshape(b, s, _TOP_K)
    return (top_k_weights, top_k_indices, aux[0, 0])


# speedup vs baseline: 1.9531x; 1.4810x over previous
import jax
import jax.numpy as jnp
from jax.experimental import pallas as pl

_N_EXPERTS = 8
_TOP_K = 2
_LB_WEIGHT = 0.01


def _router_body(x_ref, gw_ref, wt_ref, it_ref, facc_ref, pacc_ref, aux_ref):
    i = pl.program_id(0)
    nsteps = pl.num_programs(0)
    xb = x_ref[...]
    gw = gw_ref[...]
    l = jax.lax.dot_general(
        gw, xb, (((1,), (1,)), ((), ())),
        preferred_element_type=jnp.float32)
    blk = l.shape[1]
    e_iota = jax.lax.broadcasted_iota(jnp.int32, (_N_EXPERTS, blk), 0)

    m1 = jnp.max(l, axis=0, keepdims=True)
    i1 = jnp.min(jnp.where(l == m1, e_iota, _N_EXPERTS), axis=0, keepdims=True)
    lm = jnp.where(e_iota == i1, -jnp.inf, l)
    m2 = jnp.max(lm, axis=0, keepdims=True)
    i2 = jnp.min(jnp.where(lm == m2, e_iota, _N_EXPERTS), axis=0, keepdims=True)

    r = jnp.exp(m2 - m1)
    w1 = 1.0 / (1.0 + r)
    w2 = r / (1.0 + r)
    wt_ref[...] = jnp.concatenate([w1, w2], axis=0)
    it_ref[...] = jnp.concatenate([i1, i2], axis=0)

    t = jnp.exp(l - m1)
    denom = jnp.sum(t, axis=0, keepdims=True)
    probs = t / denom
    pc = jnp.sum(probs, axis=1, keepdims=True)
    mask = (e_iota == i1) | (e_iota == i2)
    fc = jnp.sum(jnp.where(mask, 1.0, 0.0), axis=1, keepdims=True)

    @pl.when(i == 0)
    def _init():
        facc_ref[...] = jnp.zeros_like(facc_ref)
        pacc_ref[...] = jnp.zeros_like(pacc_ref)

    facc_ref[...] += jnp.broadcast_to(fc, facc_ref.shape)
    pacc_ref[...] += jnp.broadcast_to(pc, pacc_ref.shape)

    @pl.when(i == nsteps - 1)
    def _fin():
        f = facc_ref[:, 0:1]
        p = pacc_ref[:, 0:1]
        s = jnp.sum(f * p)
        n_tok = jnp.float32(nsteps * blk)
        aux_ref[...] = (_N_EXPERTS * _LB_WEIGHT * s / (n_tok * n_tok)).reshape(1, 1)


def kernel(x, gate_w):
    b, s, d = x.shape
    n_tok = b * s
    xf = x.reshape(n_tok, d)
    blk = 1024
    grid = n_tok // blk

    out_shapes = (
        jax.ShapeDtypeStruct((_TOP_K, n_tok), jnp.float32),
        jax.ShapeDtypeStruct((_TOP_K, n_tok), jnp.int32),
        jax.ShapeDtypeStruct((_N_EXPERTS, 128), jnp.float32),
        jax.ShapeDtypeStruct((_N_EXPERTS, 128), jnp.float32),
        jax.ShapeDtypeStruct((1, 1), jnp.float32),
    )
    wt, it, _, _, aux = pl.pallas_call(
        _router_body,
        grid=(grid,),
        in_specs=[
            pl.BlockSpec((blk, d), lambda i: (i, 0)),
            pl.BlockSpec((_N_EXPERTS, d), lambda i: (0, 0)),
        ],
        out_specs=[
            pl.BlockSpec((_TOP_K, blk), lambda i: (0, i)),
            pl.BlockSpec((_TOP_K, blk), lambda i: (0, i)),
            pl.BlockSpec((_N_EXPERTS, 128), lambda i: (0, 0)),
            pl.BlockSpec((_N_EXPERTS, 128), lambda i: (0, 0)),
            pl.BlockSpec((1, 1), lambda i: (0, 0)),
        ],
        out_shape=out_shapes,
    )(xf, gate_w)

    top_k_weights = wt.T.reshape(b, s, _TOP_K)
    top_k_indices = it.T.reshape(b, s, _TOP_K)
    return (top_k_weights, top_k_indices, aux[0, 0])
